# baseline (speedup 1.0000x reference)
import functools

import jax
import jax.numpy as jnp
from jax import lax
from jax.experimental import pallas as pl
from jax.experimental.pallas import tpu as pltpu
from jax.experimental.pallas import tpu_sc as plsc

N = 10000
E = 320000
D = 128

NC = 2
NS = 16
NW = NC * NS

CHUNK = 128

K_SPLIT = 2
EH = E // K_SPLIT
EPW_H = EH // NW
CPW_H = EPW_H // CHUNK
TAIL_H = EPW_H - CPW_H * CHUNK
TAIL_RD = 0 if TAIL_H == 0 else 16 * ((TAIL_H + 15) // 16)

EPW = E // NW
CPW = EPW // CHUNK
TAIL = EPW - CPW * CHUNK

N_PAD = 10240
RPS = N_PAD // NS

BLK_E = 2000
BLK_N = 1000


def _mesh():
    return plsc.VectorSubcoreMesh(core_axis_name="c", subcore_axis_name="s")


def _sc_gather(xa, xb, row, col, b1, e0):
    @functools.partial(
        pl.kernel,
        out_type=jax.ShapeDtypeStruct((EH, D), jnp.float32),
        mesh=_mesh(),
        scratch_types=[
            pltpu.VMEM((CHUNK,), jnp.int32),
            pltpu.VMEM((CHUNK,), jnp.int32),
            pltpu.VMEM((CHUNK,), jnp.int32),
            pltpu.VMEM((CHUNK,), jnp.int32),
            pltpu.VMEM((CHUNK, D), jnp.float32),
            pltpu.VMEM((CHUNK, D), jnp.float32),
            pltpu.VMEM((CHUNK, D), jnp.float32),
            pltpu.VMEM((CHUNK, D), jnp.float32),
            pltpu.VMEM((CHUNK, D), jnp.float32),
            pltpu.VMEM((CHUNK, D), jnp.float32),
            pltpu.VMEM((TAIL_RD,), jnp.int32),
            pltpu.VMEM((TAIL_RD,), jnp.int32),
            pltpu.VMEM((TAIL_RD, D), jnp.float32),
            pltpu.VMEM((TAIL_RD, D), jnp.float32),
            pltpu.VMEM((D,), jnp.float32),
            pltpu.SemaphoreType.DMA,
            pltpu.SemaphoreType.DMA,
            pltpu.SemaphoreType.DMA,
            pltpu.SemaphoreType.DMA,
            pltpu.SemaphoreType.DMA,
            pltpu.SemaphoreType.DMA,
        ],
    )
    def k(xa_h, xb_h, row_h, col_h, b1_h, g_h,
          ridx0, ridx1, cidx0, cidx1, bufa0, bufa1, bufb0, bufb1,
          bufo0, bufo1, ridx_t, cidx_t, bufa_t, bufb_t, b1v,
          sga0, sga1, sgb0, sgb1, so0, so1):
        c = lax.axis_index("c")
        s = lax.axis_index("s")
        wid = s * NC + c
        base_g = wid * EPW_H
        base_e = e0 + base_g
        ridx = [ridx0, ridx1]
        cidx = [cidx0, cidx1]
        bufa = [bufa0, bufa1]
        bufb = [bufb0, bufb1]
        bufo = [bufo0, bufo1]
        sga = [sga0, sga1]
        sgb = [sgb0, sgb1]
        so = [so0, so1]

        pltpu.sync_copy(b1_h, b1v)
        b1r = [b1v[pl.ds(k8 * 16, 16)] for k8 in range(8)]

        pltpu.sync_copy(row_h.at[pl.ds(base_e, CHUNK)], ridx0)
        pltpu.sync_copy(col_h.at[pl.ds(base_e, CHUNK)], cidx0)
        pltpu.async_copy(xa_h.at[ridx0], bufa0, sga0)
        pltpu.async_copy(xb_h.at[cidx0], bufb0, sgb0)

        def do_chunk(j, b):
            nb = 1 - b

            @pl.when(j + 1 < CPW_H)
            def _():
                off = base_e + (j + 1) * CHUNK
                pltpu.sync_copy(row_h.at[pl.ds(off, CHUNK)], ridx[nb])
                pltpu.sync_copy(col_h.at[pl.ds(off, CHUNK)], cidx[nb])
                pltpu.async_copy(xa_h.at[ridx[nb]], bufa[nb], sga[nb])
                pltpu.async_copy(xb_h.at[cidx[nb]], bufb[nb], sgb[nb])

            pltpu.make_async_copy(xa_h.at[ridx[b]], bufa[b], sga[b]).wait()
            pltpu.make_async_copy(xb_h.at[cidx[b]], bufb[b], sgb[b]).wait()

            @pl.when(j >= 2)
            def _():
                pltpu.make_async_copy(
                    bufo[b], g_h.at[pl.ds(0, CHUNK)], so[b]).wait()

            def add_row(r, carry2):
                for k8 in range(8):
                    sl = pl.ds(k8 * 16, 16)
                    bufo[b][r, sl] = jnp.maximum(
                        bufa[b][r, sl] + bufb[b][r, sl] + b1r[k8], 0.0)
                return carry2

            lax.fori_loop(0, CHUNK, add_row, 0)
            pltpu.async_copy(
                bufo[b], g_h.at[pl.ds(base_g + j * CHUNK, CHUNK)], so[b])

        def body(j2, carry):
            for b in range(2):
                do_chunk(j2 * 2 + b, b)
            return carry

        lax.fori_loop(0, CPW_H // 2, body, 0)
        if CPW_H % 2:
            do_chunk(CPW_H - 1, (CPW_H - 1) % 2)
        for b in range(2):
            pltpu.make_async_copy(bufo[b], g_h.at[pl.ds(0, CHUNK)], so[b]).wait()

        if TAIL_RD:
            off_t = EPW_H - TAIL_RD
            pltpu.sync_copy(row_h.at[pl.ds(base_e + off_t, TAIL_RD)], ridx_t)
            pltpu.sync_copy(col_h.at[pl.ds(base_e + off_t, TAIL_RD)], cidx_t)
            pltpu.async_copy(xa_h.at[ridx_t], bufa_t, sga0).wait()
            pltpu.async_copy(xb_h.at[cidx_t], bufb_t, sgb0).wait()

            def add_row_t(r, carry2):
                for k8 in range(8):
                    sl = pl.ds(k8 * 16, 16)
                    bufa_t[r, sl] = jnp.maximum(
                        bufa_t[r, sl] + bufb_t[r, sl] + b1r[k8], 0.0)
                return carry2

            lax.fori_loop(0, TAIL_RD, add_row_t, 0)
            pltpu.sync_copy(bufa_t, g_h.at[pl.ds(base_g + off_t, TAIL_RD)])

    return k(xa, xb, row, col, b1)


def _sc_scatter(ef, row):
    @functools.partial(
        pl.kernel,
        out_type=jax.ShapeDtypeStruct((NC, N_PAD, D), jnp.float32),
        mesh=_mesh(),
        scratch_types=[
            pltpu.VMEM_SHARED((N_PAD, D), jnp.float32),
            pltpu.VMEM((CHUNK, D), jnp.float32),
            pltpu.VMEM((CHUNK, D), jnp.float32),
            pltpu.VMEM((CHUNK,), jnp.int32),
            pltpu.VMEM((CHUNK,), jnp.int32),
            pltpu.VMEM((TAIL, D), jnp.float32),
            pltpu.VMEM((TAIL,), jnp.int32),
            pltpu.SemaphoreType.DMA,
            pltpu.SemaphoreType.DMA,
        ],
    )
    def k(ef_h, row_h, out_h, acc, efv0, efv1, ridx0, ridx1,
          efv_t, ridx_t, se0, se1):
        c = lax.axis_index("c")
        s = lax.axis_index("s")
        wid = s * NC + c
        base_w = wid * EPW
        efv = [efv0, efv1]
        ridx = [ridx0, ridx1]
        se = [se0, se1]

        def zrow(r, carry):
            for k8 in range(8):
                efv0[r, pl.ds(k8 * 16, 16)] = jnp.zeros((16,), jnp.float32)
            return carry

        lax.fori_loop(0, CHUNK, zrow, 0)

        def zcp(t, carry):
            pltpu.sync_copy(efv0, acc.at[pl.ds(s * RPS + t * CHUNK, CHUNK)])
            return carry

        lax.fori_loop(0, RPS // CHUNK, zcp, 0)
        plsc.subcore_barrier()

        pltpu.sync_copy(row_h.at[pl.ds(base_w, CHUNK)], ridx0)
        pltpu.async_copy(ef_h.at[pl.ds(base_w, CHUNK)], efv0, se0)

        def chunk_body(j2, carry):
            for b in range(2):
                j = j2 * 2 + b
                nb = 1 - b

                @pl.when(j + 1 < CPW)
                def _():
                    off = base_w + (j + 1) * CHUNK
                    pltpu.sync_copy(row_h.at[pl.ds(off, CHUNK)], ridx[nb])
                    pltpu.async_copy(ef_h.at[pl.ds(off, CHUNK)], efv[nb], se[nb])

                pltpu.make_async_copy(
                    ef_h.at[pl.ds(0, CHUNK)], efv[b], se[b]).wait()
                pltpu.sync_copy(efv[b], acc.at[ridx[b]], add=True)
            return carry

        lax.fori_loop(0, CPW // 2, chunk_body, 0)

        off_t = base_w + CPW * CHUNK
        pltpu.sync_copy(row_h.at[pl.ds(off_t, TAIL)], ridx_t)
        pltpu.sync_copy(ef_h.at[pl.ds(off_t, TAIL)], efv_t)
        pltpu.sync_copy(efv_t, acc.at[ridx_t], add=True)
        plsc.subcore_barrier()

        def wcp(t, carry):
            r0 = s * RPS + t * CHUNK
            pltpu.sync_copy(acc.at[pl.ds(r0, CHUNK)], out_h.at[c, pl.ds(r0, CHUNK)])
            return carry

        lax.fori_loop(0, RPS // CHUNK, wcp, 0)

    return k(ef, row)


def _pre_body(x_ref, wa_ref, wb_ref, xa_ref, xb_ref):
    xa_ref[...] = jnp.dot(x_ref[...], wa_ref[...],
                          preferred_element_type=jnp.float32)
    xb_ref[...] = jnp.dot(x_ref[...], wb_ref[...],
                          preferred_element_type=jnp.float32)


def _precompute(x, w1a, w1b):
    grid = (N // BLK_N,)
    return pl.pallas_call(
        _pre_body,
        grid=grid,
        in_specs=[
            pl.BlockSpec((BLK_N, D), lambda i: (i, 0)),
            pl.BlockSpec((D, D), lambda i: (0, 0)),
            pl.BlockSpec((D, D), lambda i: (0, 0)),
        ],
        out_specs=[
            pl.BlockSpec((BLK_N, D), lambda i: (i, 0)),
            pl.BlockSpec((BLK_N, D), lambda i: (i, 0)),
        ],
        out_shape=[
            jax.ShapeDtypeStruct((N, D), jnp.float32),
            jax.ShapeDtypeStruct((N, D), jnp.float32),
        ],
    )(x, w1a, w1b)


def _edge_body(g_ref, m_ref, w_ref, b_ref, e_ref, o_ref):
    h = jnp.dot(g_ref[...].astype(jnp.bfloat16),
                w_ref[...].astype(jnp.bfloat16),
                preferred_element_type=jnp.float32)
    h = jnp.maximum(h + b_ref[...], 0.0)
    o_ref[...] = h * m_ref[...]


def _edge_body_first(g_ref, m_ref, w_ref, b_ref, o_ref):
    _edge_body(g_ref, m_ref, w_ref, b_ref, None, o_ref)


def _edge_mlp_range(g_h, mask, w2, b2r, ef_prev, h_idx):
    nblk = EH // BLK_E
    off = h_idx * nblk
    common = dict(
        grid=(nblk,),
        out_specs=pl.BlockSpec((BLK_E, D), lambda i, o=off: (i + o, 0)),
        out_shape=jax.ShapeDtypeStruct((E, D), jnp.float32),
    )
    in_specs = [
        pl.BlockSpec((BLK_E, D), lambda i: (i, 0)),
        pl.BlockSpec((BLK_E, 1), lambda i, o=off: (i + o, 0)),
        pl.BlockSpec((D, D), lambda i: (0, 0)),
        pl.BlockSpec((1, D), lambda i: (0, 0)),
    ]
    if ef_prev is None:
        return pl.pallas_call(_edge_body_first, in_specs=in_specs,
                              **common)(g_h, mask, w2, b2r)
    in_specs.append(pl.BlockSpec((8, D), lambda i: (0, 0)))
    return pl.pallas_call(_edge_body, in_specs=in_specs,
                          input_output_aliases={4: 0},
                          **common)(g_h, mask, w2, b2r, ef_prev)


def _node_body(x_ref, p_ref, wa_ref, wb_ref, b1_ref, w2_ref, b2_ref, o_ref):
    agg = p_ref[0] + p_ref[1]
    n1 = (jnp.dot(x_ref[...], wa_ref[...], preferred_element_type=jnp.float32)
          + jnp.dot(agg, wb_ref[...], preferred_element_type=jnp.float32)
          + b1_ref[...])
    n1 = jnp.maximum(n1, 0.0)
    o_ref[...] = (jnp.dot(n1, w2_ref[...], preferred_element_type=jnp.float32)
                  + b2_ref[...] + x_ref[...])


def _node_mlp(x, parts, wn1a, wn1b, bn1r, wn2, bn2r):
    grid = (N // BLK_N,)
    return pl.pallas_call(
        _node_body,
        grid=grid,
        in_specs=[
            pl.BlockSpec((BLK_N, D), lambda i: (i, 0)),
            pl.BlockSpec((NC, BLK_N, D), lambda i: (0, i, 0)),
            pl.BlockSpec((D, D), lambda i: (0, 0)),
            pl.BlockSpec((D, D), lambda i: (0, 0)),
            pl.BlockSpec((1, D), lambda i: (0, 0)),
            pl.BlockSpec((D, D), lambda i: (0, 0)),
            pl.BlockSpec((1, D), lambda i: (0, 0)),
        ],
        out_specs=pl.BlockSpec((BLK_N, D), lambda i: (i, 0)),
        out_shape=jax.ShapeDtypeStruct((N, D), jnp.float32),
    )(x, parts, wn1a, wn1b, bn1r, wn2, bn2r)


def kernel(x, edge_index, edge_mask, W1, b1, W2, b2, Wn1, bn1, Wn2, bn2):
    row = edge_index[0]
    col = edge_index[1]

    w1a, w1b = W1[:D], W1[D:]
    wn1a, wn1b = Wn1[:D], Wn1[D:]

    xa, xb = _precompute(x, w1a, w1b)
    gs = [_sc_gather(xa, xb, row, col, b1, h * EH) for h in range(K_SPLIT)]

    b2r = b2.reshape(1, D)
    ef = None
    for h in range(K_SPLIT):
        ef = _edge_mlp_range(gs[h], edge_mask, W2, b2r, ef, h)

    parts = _sc_scatter(ef, row)
    out = _node_mlp(x, parts, wn1a, wn1b, bn1.reshape(1, D),
                    Wn2, bn2.reshape(1, D))
    return out, ef


# speedup vs baseline: 1.0182x; 1.0182x over previous
import functools

import jax
import jax.numpy as jnp
from jax import lax
from jax.experimental import pallas as pl
from jax.experimental.pallas import tpu as pltpu
from jax.experimental.pallas import tpu_sc as plsc

N = 10000
E = 320000
D = 128

NC = 2
NS = 16
NW = NC * NS

CHUNK = 128

K_SPLIT = 5
EH = E // K_SPLIT
EPW_H = EH // NW
CPW_H = EPW_H // CHUNK
TAIL_H = EPW_H - CPW_H * CHUNK
TAIL_RD = 0 if TAIL_H == 0 else 16 * ((TAIL_H + 15) // 16)

EPW = E // NW
CPW = EPW // CHUNK
TAIL = EPW - CPW * CHUNK

N_PAD = 10240
RPS = N_PAD // NS

BLK_E = 2560
BLK_N = 1000


def _mesh():
    return plsc.VectorSubcoreMesh(core_axis_name="c", subcore_axis_name="s")


def _sc_gather(xa, xb, row, col, b1, e0):
    @functools.partial(
        pl.kernel,
        out_type=jax.ShapeDtypeStruct((EH, D), jnp.float32),
        mesh=_mesh(),
        scratch_types=[
            pltpu.VMEM((CHUNK,), jnp.int32),
            pltpu.VMEM((CHUNK,), jnp.int32),
            pltpu.VMEM((CHUNK,), jnp.int32),
            pltpu.VMEM((CHUNK,), jnp.int32),
            pltpu.VMEM((CHUNK, D), jnp.float32),
            pltpu.VMEM((CHUNK, D), jnp.float32),
            pltpu.VMEM((CHUNK, D), jnp.float32),
            pltpu.VMEM((CHUNK, D), jnp.float32),
            pltpu.VMEM((CHUNK, D), jnp.float32),
            pltpu.VMEM((CHUNK, D), jnp.float32),
            pltpu.VMEM((TAIL_RD,), jnp.int32),
            pltpu.VMEM((TAIL_RD,), jnp.int32),
            pltpu.VMEM((TAIL_RD, D), jnp.float32),
            pltpu.VMEM((TAIL_RD, D), jnp.float32),
            pltpu.VMEM((D,), jnp.float32),
            pltpu.SemaphoreType.DMA,
            pltpu.SemaphoreType.DMA,
            pltpu.SemaphoreType.DMA,
            pltpu.SemaphoreType.DMA,
            pltpu.SemaphoreType.DMA,
            pltpu.SemaphoreType.DMA,
        ],
    )
    def k(xa_h, xb_h, row_h, col_h, b1_h, g_h,
          ridx0, ridx1, cidx0, cidx1, bufa0, bufa1, bufb0, bufb1,
          bufo0, bufo1, ridx_t, cidx_t, bufa_t, bufb_t, b1v,
          sga0, sga1, sgb0, sgb1, so0, so1):
        c = lax.axis_index("c")
        s = lax.axis_index("s")
        wid = s * NC + c
        base_g = wid * EPW_H
        base_e = e0 + base_g
        ridx = [ridx0, ridx1]
        cidx = [cidx0, cidx1]
        bufa = [bufa0, bufa1]
        bufb = [bufb0, bufb1]
        bufo = [bufo0, bufo1]
        sga = [sga0, sga1]
        sgb = [sgb0, sgb1]
        so = [so0, so1]

        pltpu.sync_copy(b1_h, b1v)
        b1r = [b1v[pl.ds(k8 * 16, 16)] for k8 in range(8)]

        pltpu.sync_copy(row_h.at[pl.ds(base_e, CHUNK)], ridx0)
        pltpu.sync_copy(col_h.at[pl.ds(base_e, CHUNK)], cidx0)
        pltpu.async_copy(xa_h.at[ridx0], bufa0, sga0)
        pltpu.async_copy(xb_h.at[cidx0], bufb0, sgb0)

        def do_chunk(j, b):
            nb = 1 - b

            @pl.when(j + 1 < CPW_H)
            def _():
                off = base_e + (j + 1) * CHUNK
                pltpu.sync_copy(row_h.at[pl.ds(off, CHUNK)], ridx[nb])
                pltpu.sync_copy(col_h.at[pl.ds(off, CHUNK)], cidx[nb])
                pltpu.async_copy(xa_h.at[ridx[nb]], bufa[nb], sga[nb])
                pltpu.async_copy(xb_h.at[cidx[nb]], bufb[nb], sgb[nb])

            pltpu.make_async_copy(xa_h.at[ridx[b]], bufa[b], sga[b]).wait()
            pltpu.make_async_copy(xb_h.at[cidx[b]], bufb[b], sgb[b]).wait()

            @pl.when(j >= 2)
            def _():
                pltpu.make_async_copy(
                    bufo[b], g_h.at[pl.ds(0, CHUNK)], so[b]).wait()

            def add_row(r, carry2):
                for k8 in range(8):
                    sl = pl.ds(k8 * 16, 16)
                    bufo[b][r, sl] = jnp.maximum(
                        bufa[b][r, sl] + bufb[b][r, sl] + b1r[k8], 0.0)
                return carry2

            lax.fori_loop(0, CHUNK, add_row, 0)
            pltpu.async_copy(
                bufo[b], g_h.at[pl.ds(base_g + j * CHUNK, CHUNK)], so[b])

        def body(j2, carry):
            for b in range(2):
                do_chunk(j2 * 2 + b, b)
            return carry

        lax.fori_loop(0, CPW_H // 2, body, 0)
        if CPW_H % 2:
            do_chunk(CPW_H - 1, (CPW_H - 1) % 2)
        for b in range(2):
            pltpu.make_async_copy(bufo[b], g_h.at[pl.ds(0, CHUNK)], so[b]).wait()

        if TAIL_RD:
            off_t = EPW_H - TAIL_RD
            pltpu.sync_copy(row_h.at[pl.ds(base_e + off_t, TAIL_RD)], ridx_t)
            pltpu.sync_copy(col_h.at[pl.ds(base_e + off_t, TAIL_RD)], cidx_t)
            pltpu.async_copy(xa_h.at[ridx_t], bufa_t, sga0).wait()
            pltpu.async_copy(xb_h.at[cidx_t], bufb_t, sgb0).wait()

            def add_row_t(r, carry2):
                for k8 in range(8):
                    sl = pl.ds(k8 * 16, 16)
                    bufa_t[r, sl] = jnp.maximum(
                        bufa_t[r, sl] + bufb_t[r, sl] + b1r[k8], 0.0)
                return carry2

            lax.fori_loop(0, TAIL_RD, add_row_t, 0)
            pltpu.sync_copy(bufa_t, g_h.at[pl.ds(base_g + off_t, TAIL_RD)])

    return k(xa, xb, row, col, b1)


def _sc_scatter(ef, row):
    @functools.partial(
        pl.kernel,
        out_type=jax.ShapeDtypeStruct((NC, N_PAD, D), jnp.float32),
        mesh=_mesh(),
        scratch_types=[
            pltpu.VMEM_SHARED((N_PAD, D), jnp.float32),
            pltpu.VMEM((CHUNK, D), jnp.float32),
            pltpu.VMEM((CHUNK, D), jnp.float32),
            pltpu.VMEM((CHUNK,), jnp.int32),
            pltpu.VMEM((CHUNK,), jnp.int32),
            pltpu.VMEM((TAIL, D), jnp.float32),
            pltpu.VMEM((TAIL,), jnp.int32),
            pltpu.SemaphoreType.DMA,
            pltpu.SemaphoreType.DMA,
        ],
    )
    def k(ef_h, row_h, out_h, acc, efv0, efv1, ridx0, ridx1,
          efv_t, ridx_t, se0, se1):
        c = lax.axis_index("c")
        s = lax.axis_index("s")
        wid = s * NC + c
        base_w = wid * EPW
        efv = [efv0, efv1]
        ridx = [ridx0, ridx1]
        se = [se0, se1]

        def zrow(r, carry):
            for k8 in range(8):
                efv0[r, pl.ds(k8 * 16, 16)] = jnp.zeros((16,), jnp.float32)
            return carry

        lax.fori_loop(0, CHUNK, zrow, 0)

        def zcp(t, carry):
            pltpu.sync_copy(efv0, acc.at[pl.ds(s * RPS + t * CHUNK, CHUNK)])
            return carry

        lax.fori_loop(0, RPS // CHUNK, zcp, 0)
        plsc.subcore_barrier()

        pltpu.sync_copy(row_h.at[pl.ds(base_w, CHUNK)], ridx0)
        pltpu.async_copy(ef_h.at[pl.ds(base_w, CHUNK)], efv0, se0)

        def chunk_body(j2, carry):
            for b in range(2):
                j = j2 * 2 + b
                nb = 1 - b

                @pl.when(j + 1 < CPW)
                def _():
                    off = base_w + (j + 1) * CHUNK
                    pltpu.sync_copy(row_h.at[pl.ds(off, CHUNK)], ridx[nb])
                    pltpu.async_copy(ef_h.at[pl.ds(off, CHUNK)], efv[nb], se[nb])

                pltpu.make_async_copy(
                    ef_h.at[pl.ds(0, CHUNK)], efv[b], se[b]).wait()
                pltpu.sync_copy(efv[b], acc.at[ridx[b]], add=True)
            return carry

        lax.fori_loop(0, CPW // 2, chunk_body, 0)

        off_t = base_w + CPW * CHUNK
        pltpu.sync_copy(row_h.at[pl.ds(off_t, TAIL)], ridx_t)
        pltpu.sync_copy(ef_h.at[pl.ds(off_t, TAIL)], efv_t)
        pltpu.sync_copy(efv_t, acc.at[ridx_t], add=True)
        plsc.subcore_barrier()

        def wcp(t, carry):
            r0 = s * RPS + t * CHUNK
            pltpu.sync_copy(acc.at[pl.ds(r0, CHUNK)], out_h.at[c, pl.ds(r0, CHUNK)])
            return carry

        lax.fori_loop(0, RPS // CHUNK, wcp, 0)

    return k(ef, row)


def _pre_body(x_ref, wa_ref, wb_ref, xa_ref, xb_ref):
    xa_ref[...] = jnp.dot(x_ref[...], wa_ref[...],
                          preferred_element_type=jnp.float32)
    xb_ref[...] = jnp.dot(x_ref[...], wb_ref[...],
                          preferred_element_type=jnp.float32)


def _precompute(x, w1a, w1b):
    grid = (N // BLK_N,)
    return pl.pallas_call(
        _pre_body,
        grid=grid,
        in_specs=[
            pl.BlockSpec((BLK_N, D), lambda i: (i, 0)),
            pl.BlockSpec((D, D), lambda i: (0, 0)),
            pl.BlockSpec((D, D), lambda i: (0, 0)),
        ],
        out_specs=[
            pl.BlockSpec((BLK_N, D), lambda i: (i, 0)),
            pl.BlockSpec((BLK_N, D), lambda i: (i, 0)),
        ],
        out_shape=[
            jax.ShapeDtypeStruct((N, D), jnp.float32),
            jax.ShapeDtypeStruct((N, D), jnp.float32),
        ],
    )(x, w1a, w1b)


def _edge_body(g_ref, m_ref, w_ref, b_ref, e_ref, o_ref):
    h = jnp.dot(g_ref[...].astype(jnp.bfloat16),
                w_ref[...].astype(jnp.bfloat16),
                preferred_element_type=jnp.float32)
    h = jnp.maximum(h + b_ref[...], 0.0)
    o_ref[...] = h * m_ref[...]


def _edge_body_first(g_ref, m_ref, w_ref, b_ref, o_ref):
    _edge_body(g_ref, m_ref, w_ref, b_ref, None, o_ref)


def _edge_mlp_range(g_h, mask, w2, b2r, ef_prev, h_idx):
    nblk = EH // BLK_E
    off = h_idx * nblk
    common = dict(
        grid=(nblk,),
        out_specs=pl.BlockSpec((BLK_E, D), lambda i, o=off: (i + o, 0)),
        out_shape=jax.ShapeDtypeStruct((E, D), jnp.float32),
    )
    in_specs = [
        pl.BlockSpec((BLK_E, D), lambda i: (i, 0)),
        pl.BlockSpec((BLK_E, 1), lambda i, o=off: (i + o, 0)),
        pl.BlockSpec((D, D), lambda i: (0, 0)),
        pl.BlockSpec((1, D), lambda i: (0, 0)),
    ]
    if ef_prev is None:
        return pl.pallas_call(_edge_body_first, in_specs=in_specs,
                              **common)(g_h, mask, w2, b2r)
    in_specs.append(pl.BlockSpec((8, D), lambda i: (0, 0)))
    return pl.pallas_call(_edge_body, in_specs=in_specs,
                          input_output_aliases={4: 0},
                          **common)(g_h, mask, w2, b2r, ef_prev)


def _node_body(x_ref, p_ref, wa_ref, wb_ref, b1_ref, w2_ref, b2_ref, o_ref):
    agg = p_ref[0] + p_ref[1]
    n1 = (jnp.dot(x_ref[...], wa_ref[...], preferred_element_type=jnp.float32)
          + jnp.dot(agg, wb_ref[...], preferred_element_type=jnp.float32)
          + b1_ref[...])
    n1 = jnp.maximum(n1, 0.0)
    o_ref[...] = (jnp.dot(n1, w2_ref[...], preferred_element_type=jnp.float32)
                  + b2_ref[...] + x_ref[...])


def _node_mlp(x, parts, wn1a, wn1b, bn1r, wn2, bn2r):
    grid = (N // BLK_N,)
    return pl.pallas_call(
        _node_body,
        grid=grid,
        in_specs=[
            pl.BlockSpec((BLK_N, D), lambda i: (i, 0)),
            pl.BlockSpec((NC, BLK_N, D), lambda i: (0, i, 0)),
            pl.BlockSpec((D, D), lambda i: (0, 0)),
            pl.BlockSpec((D, D), lambda i: (0, 0)),
            pl.BlockSpec((1, D), lambda i: (0, 0)),
            pl.BlockSpec((D, D), lambda i: (0, 0)),
            pl.BlockSpec((1, D), lambda i: (0, 0)),
        ],
        out_specs=pl.BlockSpec((BLK_N, D), lambda i: (i, 0)),
        out_shape=jax.ShapeDtypeStruct((N, D), jnp.float32),
    )(x, parts, wn1a, wn1b, bn1r, wn2, bn2r)


def kernel(x, edge_index, edge_mask, W1, b1, W2, b2, Wn1, bn1, Wn2, bn2):
    row = edge_index[0]
    col = edge_index[1]

    w1a, w1b = W1[:D], W1[D:]
    wn1a, wn1b = Wn1[:D], Wn1[D:]

    xa, xb = _precompute(x, w1a, w1b)
    gs = [_sc_gather(xa, xb, row, col, b1, h * EH) for h in range(K_SPLIT)]

    b2r = b2.reshape(1, D)
    ef = None
    for h in range(K_SPLIT):
        ef = _edge_mlp_range(gs[h], edge_mask, W2, b2r, ef, h)

    parts = _sc_scatter(ef, row)
    out = _node_mlp(x, parts, wn1a, wn1b, bn1.reshape(1, D),
                    Wn2, bn2.reshape(1, D))
    return out, ef
